# baseline (speedup 1.0000x reference)
import functools

import jax
import jax.numpy as jnp
from jax import lax
from jax.experimental import pallas as pl
from jax.experimental.pallas import tpu as pltpu
from jax.experimental.pallas import tpu_sc as plsc

_ROWS, _COLS, _DIM = 16384, 200, 3
_N_IN = _ROWS * _COLS
_NC, _NS, _L = 2, 16, 16
_NW = _NC * _NS
_PER_W = _N_IN // _NW
_BLK = 10_240
_NBLK = _PER_W // _BLK
_NBUF = 3
_CHUNKS = _BLK // _L
_TS, _TL = 8, 128


def _make_kernel():
    mesh = plsc.VectorSubcoreMesh(core_axis_name="c", subcore_axis_name="s")

    @functools.partial(
        pl.kernel,
        mesh=mesh,
        out_type=jax.ShapeDtypeStruct((_DIM * _N_IN,), jnp.float32),
        compiler_params=pltpu.CompilerParams(needs_layout_passes=False),
        scratch_types=(
            [pltpu.VMEM((2 * _DIM * _L,), jnp.float32)]
            + [pltpu.VMEM((_BLK,), jnp.int32)] * _NBUF
            + [pltpu.VMEM((_BLK,), jnp.float32)] * (3 * _NBUF)
            + [pltpu.SemaphoreType.DMA] * (2 * _NBUF)
        ),
    )
    def emb_kernel(x_hbm, wsplat_hbm, out_hbm, ws_v, *scratch):
        xbs = scratch[:_NBUF]
        obs = tuple(
            scratch[_NBUF + 3 * u:_NBUF + 3 * u + 3] for u in range(_NBUF))
        sins = scratch[4 * _NBUF:5 * _NBUF]
        souts = scratch[5 * _NBUF:6 * _NBUF]
        wid = lax.axis_index("s") * _NC + lax.axis_index("c")
        base = wid * _PER_W

        pltpu.sync_copy(wsplat_hbm, ws_v)
        ws0 = tuple(ws_v[pl.ds(v * _L, _L)] for v in range(2 * _DIM))

        def start_in(b):
            ib = pl.multiple_of(base + b * _BLK, 8)
            return pltpu.async_copy(
                x_hbm.at[pl.ds(ib, _BLK)], xbs[b % _NBUF], sins[b % _NBUF])

        in_copies = [None] * _NBLK
        out_copies = [None] * _NBLK
        for b in range(_NBUF):
            in_copies[b] = start_in(b)
        for b in range(_NBLK):
            buf = b % _NBUF
            in_copies[b].wait()
            if b >= _NBUF:
                for c in out_copies[b - _NBUF]:
                    c.wait()

            xb = xbs[buf]
            ob = obs[buf]

            def chunk(t, ws):
                (w00, w01, w02, w10, w11, w12) = ws
                w0 = (w00, w01, w02)
                w1 = (w10, w11, w12)
                off = t * _L
                m = xb[pl.ds(off, _L)] > 0
                for k in range(_DIM):
                    ob[k][pl.ds(off, _L)] = jnp.where(m, w1[k], w0[k])
                return ws

            lax.fori_loop(0, _CHUNKS, chunk, ws0, unroll=8)

            ocs = []
            for k in range(_DIM):
                oo = pl.multiple_of(k * _N_IN + base + b * _BLK, 8)
                ocs.append(pltpu.async_copy(
                    ob[k], out_hbm.at[pl.ds(oo, _BLK)], souts[buf]))
            out_copies[b] = ocs
            if b + _NBUF < _NBLK:
                in_copies[b + _NBUF] = start_in(b + _NBUF)
        for b in range(_NBLK - _NBUF, _NBLK):
            for c in out_copies[b]:
                c.wait()

    return emb_kernel


_emb_kernel = _make_kernel()


def kernel(x, emb_weight):
    ws = emb_weight.astype(jnp.float32) * jnp.float32(3.0) ** jnp.float32(0.5)
    wsplat = jnp.broadcast_to(ws.reshape(2 * _DIM, 1), (2 * _DIM, _L)).reshape(-1)
    xraw = x.reshape(_ROWS // _TL, _TL, _COLS // _TS, _TS)
    xraw = xraw.transpose(2, 0, 3, 1).reshape(-1)
    out_flat = _emb_kernel(xraw, wsplat)
    o5 = out_flat.reshape(_DIM, _COLS // _TS, _ROWS // _TL, _TS, _TL)
    return o5.transpose(2, 4, 1, 3, 0).reshape(_ROWS, _COLS, _DIM)


# speedup vs baseline: 1.5880x; 1.5880x over previous
import functools

import jax
import jax.numpy as jnp
from jax import lax
from jax.experimental import pallas as pl
from jax.experimental.pallas import tpu as pltpu
from jax.experimental.pallas import tpu_sc as plsc

_ROWS, _COLS, _DIM = 16384, 200, 3
_N_IN = _ROWS * _COLS
_NC, _NS, _L = 2, 16, 16
_NW = _NC * _NS
_PER_W = _N_IN // _NW
_BLK = 10_240
_NBLK = _PER_W // _BLK
_NBUF = 3
_CHUNKS = _BLK // _L
_TS, _TL = 8, 128


def _make_kernel():
    mesh = plsc.VectorSubcoreMesh(core_axis_name="c", subcore_axis_name="s")

    @functools.partial(
        pl.kernel,
        mesh=mesh,
        out_type=jax.ShapeDtypeStruct((_DIM * _N_IN,), jnp.float32),
        compiler_params=pltpu.CompilerParams(needs_layout_passes=False),
        scratch_types=(
            [pltpu.VMEM((2 * _DIM * _L,), jnp.float32)]
            + [pltpu.VMEM((_BLK,), jnp.int32)] * _NBUF
            + [pltpu.VMEM((_BLK,), jnp.float32)] * (3 * _NBUF)
            + [pltpu.SemaphoreType.DMA] * (2 * _NBUF)
        ),
    )
    def emb_kernel(x_hbm, wsplat_hbm, out_hbm, ws_v, *scratch):
        xbs = scratch[:_NBUF]
        obs = tuple(
            scratch[_NBUF + 3 * u:_NBUF + 3 * u + 3] for u in range(_NBUF))
        sins = scratch[4 * _NBUF:5 * _NBUF]
        souts = scratch[5 * _NBUF:6 * _NBUF]
        wid = lax.axis_index("s") * _NC + lax.axis_index("c")
        base = wid * _PER_W

        pltpu.sync_copy(wsplat_hbm, ws_v)
        ws0 = tuple(ws_v[pl.ds(v * _L, _L)] for v in range(2 * _DIM))

        def start_in(b):
            ib = pl.multiple_of(base + b * _BLK, 8)
            return pltpu.async_copy(
                x_hbm.at[pl.ds(ib, _BLK)], xbs[b % _NBUF], sins[b % _NBUF])

        in_copies = [None] * _NBLK
        out_copies = [None] * _NBLK
        for b in range(_NBUF):
            in_copies[b] = start_in(b)
        for b in range(_NBLK):
            buf = b % _NBUF
            in_copies[b].wait()
            if b >= _NBUF:
                for c in out_copies[b - _NBUF]:
                    c.wait()

            xb = xbs[buf]
            ob = obs[buf]

            @functools.partial(
                plsc.parallel_loop, 0, _CHUNKS, unroll=8, carry=ws0)
            def _(t, ws):
                (w00, w01, w02, w10, w11, w12) = ws
                w0 = (w00, w01, w02)
                w1 = (w10, w11, w12)
                off = t * _L
                m = xb[pl.ds(off, _L)] > 0
                for k in range(_DIM):
                    ob[k][pl.ds(off, _L)] = jnp.where(m, w1[k], w0[k])
                return ws

            ocs = []
            for k in range(_DIM):
                oo = pl.multiple_of(k * _N_IN + base + b * _BLK, 8)
                ocs.append(pltpu.async_copy(
                    ob[k], out_hbm.at[pl.ds(oo, _BLK)], souts[buf]))
            out_copies[b] = ocs
            if b + _NBUF < _NBLK:
                in_copies[b + _NBUF] = start_in(b + _NBUF)
        for b in range(_NBLK - _NBUF, _NBLK):
            for c in out_copies[b]:
                c.wait()

    return emb_kernel


_emb_kernel = _make_kernel()


def kernel(x, emb_weight):
    ws = emb_weight.astype(jnp.float32) * jnp.float32(3.0) ** jnp.float32(0.5)
    wsplat = jnp.broadcast_to(ws.reshape(2 * _DIM, 1), (2 * _DIM, _L)).reshape(-1)
    xraw = x.reshape(_ROWS // _TL, _TL, _COLS // _TS, _TS)
    xraw = xraw.transpose(2, 0, 3, 1).reshape(-1)
    out_flat = _emb_kernel(xraw, wsplat)
    o5 = out_flat.reshape(_DIM, _COLS // _TS, _ROWS // _TL, _TS, _TL)
    return o5.transpose(2, 4, 1, 3, 0).reshape(_ROWS, _COLS, _DIM)
